# baseline (speedup 1.0000x reference)
import functools

import jax
import jax.numpy as jnp
from jax.experimental import pallas as pl
from jax.experimental.pallas import tpu as pltpu


def _pipeline_kernel(q_hbm, z_ref, w0_ref, b0_ref, w1_ref, b1_ref, o_ref,
                     qbuf, sem, *, num_types, chunk, nchunks):
    neurons = w0_ref.shape[2]
    th = num_types * neurons

    def copy_in(c):
        return pltpu.make_async_copy(
            q_hbm.at[pl.ds(c * chunk, chunk), :], qbuf.at[c], sem.at[c])

    for c in range(nchunks):
        copy_in(c).start()

    w0r = jnp.concatenate([w0_ref[t] for t in range(num_types)], axis=1)
    w0rb = w0r.astype(jnp.bfloat16)
    b0row = jnp.concatenate([b0_ref[t:t + 1, :] for t in range(num_types)], axis=1)
    b0c = jnp.transpose(b0row)
    c_iota = jax.lax.broadcasted_iota(jnp.int32, (num_types, th), 1)
    r_iota = jax.lax.broadcasted_iota(jnp.int32, (num_types, th), 0)
    w1tile = jnp.tile(w1_ref[...], (1, num_types))
    ewt = jnp.where(c_iota // neurons == r_iota, w1tile, 0.0)
    b1 = b1_ref[0, 0]
    t_iota = jax.lax.broadcasted_iota(jnp.int32, (num_types, chunk), 0)

    for c in range(nchunks):
        copy_in(c).wait()
        qb = qbuf[c]
        pt = jax.lax.dot_general(w0rb, qb.astype(jnp.bfloat16),
                                 (((0,), (1,)), ((), ())),
                                 preferred_element_type=jnp.float32)
        ht = jnp.tanh(pt + b0c)
        st = jnp.dot(ewt, ht, preferred_element_type=jnp.float32)
        sel = jnp.where(t_iota == z_ref[pl.ds(c * chunk, chunk)][None, :], st, 0.0)
        o_ref[pl.ds(c * chunk, chunk)] = jnp.sum(sel, axis=0) + b1


def kernel(q, Z, W0, b0, W1, b1):
    n, d = q.shape
    num_types, _, neurons = W0.shape
    chunk = 1024
    nchunks = n // chunk

    b1a = jnp.full((1, 1), b1, dtype=jnp.float32)

    f = pl.pallas_call(
        functools.partial(_pipeline_kernel, num_types=num_types,
                          chunk=chunk, nchunks=nchunks),
        in_specs=[
            pl.BlockSpec(memory_space=pltpu.MemorySpace.HBM),
            pl.BlockSpec(memory_space=pltpu.MemorySpace.VMEM),
            pl.BlockSpec(memory_space=pltpu.MemorySpace.VMEM),
            pl.BlockSpec(memory_space=pltpu.MemorySpace.VMEM),
            pl.BlockSpec(memory_space=pltpu.MemorySpace.VMEM),
            pl.BlockSpec(memory_space=pltpu.MemorySpace.VMEM),
        ],
        out_specs=pl.BlockSpec(memory_space=pltpu.MemorySpace.VMEM),
        out_shape=jax.ShapeDtypeStruct((n,), jnp.float32),
        scratch_shapes=[
            pltpu.VMEM((nchunks, chunk, d), jnp.float32),
            pltpu.SemaphoreType.DMA((nchunks,)),
        ],
    )(q, Z, W0, b0, W1, b1a)

    return f


# speedup vs baseline: 1.0968x; 1.0968x over previous
import functools

import jax
import jax.numpy as jnp
from jax.experimental import pallas as pl
from jax.experimental.pallas import tpu as pltpu


def _mlp_block_kernel(q_ref, z_ref, w0_ref, b0_ref, w1_ref, b1_ref, o_ref,
                      w0r_s, b0c_s, ewt_s, *, num_types):
    neurons = w0_ref.shape[2]
    th = num_types * neurons

    @pl.when(pl.program_id(0) == 0)
    def _prep():
        w0r_s[...] = jnp.concatenate([w0_ref[t] for t in range(num_types)], axis=1)
        b0row = jnp.concatenate([b0_ref[t:t + 1, :] for t in range(num_types)], axis=1)
        b0c_s[...] = jnp.transpose(b0row)
        c_iota = jax.lax.broadcasted_iota(jnp.int32, (num_types, th), 1)
        r_iota = jax.lax.broadcasted_iota(jnp.int32, (num_types, th), 0)
        w1tile = jnp.tile(w1_ref[...], (1, num_types))
        ewt_s[...] = jnp.where(c_iota // neurons == r_iota, w1tile, 0.0)

    qb = q_ref[...]
    blk = qb.shape[0]
    pt = jax.lax.dot_general(w0r_s[...].astype(jnp.bfloat16), qb.astype(jnp.bfloat16),
                             (((0,), (1,)), ((), ())),
                             preferred_element_type=jnp.float32)
    ht = jnp.tanh(pt + b0c_s[...])
    st = jnp.dot(ewt_s[...], ht, preferred_element_type=jnp.float32)
    t_iota = jax.lax.broadcasted_iota(jnp.int32, (num_types, blk), 0)
    sel = jnp.where(t_iota == z_ref[...][None, :], st, 0.0)
    o_ref[...] = jnp.sum(sel, axis=0) + b1_ref[0, 0]


def kernel(q, Z, W0, b0, W1, b1):
    n, d = q.shape
    num_types, _, neurons = W0.shape
    th = num_types * neurons
    blk = 8192
    grid = n // blk

    b1a = jnp.full((1, 1), b1, dtype=jnp.float32)

    f = pl.pallas_call(
        functools.partial(_mlp_block_kernel, num_types=num_types),
        grid=(grid,),
        in_specs=[
            pl.BlockSpec((blk, d), lambda i: (i, 0)),
            pl.BlockSpec((blk,), lambda i: (i,)),
            pl.BlockSpec((num_types, d, neurons), lambda i: (0, 0, 0)),
            pl.BlockSpec((num_types, neurons), lambda i: (0, 0)),
            pl.BlockSpec((num_types, neurons), lambda i: (0, 0)),
            pl.BlockSpec((1, 1), lambda i: (0, 0)),
        ],
        out_specs=pl.BlockSpec((blk,), lambda i: (i,)),
        out_shape=jax.ShapeDtypeStruct((n,), jnp.float32),
        compiler_params=pltpu.CompilerParams(
            allow_input_fusion=[False, False, False, False, False, True]),
        scratch_shapes=[
            pltpu.VMEM((d, th), jnp.float32),
            pltpu.VMEM((th, 1), jnp.float32),
            pltpu.VMEM((num_types, th), jnp.float32),
        ],
    )(q, Z, W0, b0, W1, b1a)

    return f


# speedup vs baseline: 1.1594x; 1.0571x over previous
import functools

import jax
import jax.numpy as jnp
from jax.experimental import pallas as pl
from jax.experimental.pallas import tpu as pltpu


def _mlp_block_kernel(q_ref, z_ref, w0_ref, b0_ref, w1_ref, b1_ref, o_ref,
                      w0r_s, b0c_s, ewt_s, *, num_types):
    neurons = w0_ref.shape[2]
    th = num_types * neurons

    @pl.when(pl.program_id(0) == 0)
    def _prep():
        w0r_s[...] = jnp.concatenate([w0_ref[t] for t in range(num_types)], axis=1)
        b0row = jnp.concatenate([b0_ref[t:t + 1, :] for t in range(num_types)], axis=1)
        b0c_s[...] = jnp.transpose(b0row)
        c_iota = jax.lax.broadcasted_iota(jnp.int32, (num_types, th), 1)
        r_iota = jax.lax.broadcasted_iota(jnp.int32, (num_types, th), 0)
        w1tile = jnp.tile(w1_ref[...], (1, num_types))
        ewt_s[...] = jnp.where(c_iota // neurons == r_iota, w1tile, 0.0)

    qb = q_ref[...]
    blk = qb.shape[0]
    pt = jax.lax.dot_general(w0r_s[...].astype(jnp.bfloat16), qb.astype(jnp.bfloat16),
                             (((0,), (1,)), ((), ())),
                             preferred_element_type=jnp.float32)
    ht = jnp.tanh(pt + b0c_s[...])
    st = jnp.dot(ewt_s[...], ht, preferred_element_type=jnp.float32)
    t_iota = jax.lax.broadcasted_iota(jnp.int32, (num_types, blk), 0)
    sel = jnp.where(t_iota == z_ref[...][None, :], st, 0.0)
    o_ref[...] = jnp.sum(sel, axis=0) + b1_ref[0, 0]


def kernel(q, Z, W0, b0, W1, b1):
    n, d = q.shape
    num_types, _, neurons = W0.shape
    th = num_types * neurons
    blk = 4096
    grid = n // blk

    b1a = jnp.full((1, 1), b1, dtype=jnp.float32)

    f = pl.pallas_call(
        functools.partial(_mlp_block_kernel, num_types=num_types),
        grid=(grid,),
        in_specs=[
            pl.BlockSpec((blk, d), lambda i: (i, 0)),
            pl.BlockSpec((blk,), lambda i: (i,)),
            pl.BlockSpec((num_types, d, neurons), lambda i: (0, 0, 0)),
            pl.BlockSpec((num_types, neurons), lambda i: (0, 0)),
            pl.BlockSpec((num_types, neurons), lambda i: (0, 0)),
            pl.BlockSpec((1, 1), lambda i: (0, 0)),
        ],
        out_specs=pl.BlockSpec((blk,), lambda i: (i,)),
        out_shape=jax.ShapeDtypeStruct((n,), jnp.float32),
        compiler_params=pltpu.CompilerParams(
            allow_input_fusion=[False, False, False, False, False, True]),
        scratch_shapes=[
            pltpu.VMEM((d, th), jnp.float32),
            pltpu.VMEM((th, 1), jnp.float32),
            pltpu.VMEM((num_types, th), jnp.float32),
        ],
    )(q, Z, W0, b0, W1, b1a)

    return f
